# baseline (speedup 1.0000x reference)
import functools

import jax
import jax.numpy as jnp
from jax import lax
from jax.experimental import pallas as pl
from jax.experimental.pallas import tpu as pltpu
from jax.experimental.pallas import tpu_sc as plsc

NCORES = 2
NSUB = 16
LANES = 16
KE = 125


def _mesh():
    return plsc.VectorSubcoreMesh(core_axis_name="c", subcore_axis_name="s")


def _drain(src, dst, sem):
    pltpu.make_async_copy(src, dst, sem).wait()


def _make_degree(N, E):
    NR = E // KE
    WR = NR // (NCORES * NSUB)
    CH = 40
    NCHUNK = N // CH

    @functools.partial(
        pl.kernel,
        out_type=jax.ShapeDtypeStruct((NCORES * N, 128), jnp.float32),
        mesh=_mesh(),
        scratch_types=[
            pltpu.VMEM((WR, KE), jnp.int32),
            pltpu.VMEM((KE, 128), jnp.float32),
            pltpu.VMEM((CH, 128), jnp.float32),
            pltpu.VMEM_SHARED((N, 128), jnp.float32),
            pltpu.SemaphoreType.DMA,
            pltpu.SemaphoreType.DMA,
        ],
    )
    def deg_kernel(col_hbm, out_hbm, slab, ones, zbuf, acc, sem, sem2):
        c = lax.axis_index("c")
        s = lax.axis_index("s")
        w = s * NCORES + c

        def fill(r, _):
            for j in range(128 // LANES):
                sl = pl.ds(j * LANES, LANES)
                ones[r % KE, sl] = jnp.ones((LANES,), jnp.float32)
                zbuf[r % CH, sl] = jnp.zeros((LANES,), jnp.float32)
            return 0

        lax.fori_loop(0, max(KE, CH), fill, 0)
        pltpu.sync_copy(col_hbm.at[pl.ds(w * WR, WR)], slab)
        for i in range((NCHUNK + NSUB - 1) // NSUB):
            k = s + i * NSUB

            @pl.when(k < NCHUNK)
            def _():
                pltpu.sync_copy(zbuf, acc.at[pl.ds(k * CH, CH)])
        plsc.subcore_barrier()

        def batch(bi, _):
            d0 = pltpu.async_copy(ones, acc.at[slab.at[2 * bi]], sem,
                                  add=True)
            d1 = pltpu.async_copy(ones, acc.at[slab.at[2 * bi + 1]], sem2,
                                  add=True)
            d0.wait()
            d1.wait()
            return 0

        lax.fori_loop(0, WR // 2, batch, 0)
        plsc.subcore_barrier()
        for i in range((NCHUNK + NSUB - 1) // NSUB):
            k = s + i * NSUB

            @pl.when(k < NCHUNK)
            def _():
                pltpu.sync_copy(acc.at[pl.ds(k * CH, CH)],
                                out_hbm.at[pl.ds(c * N + k * CH, CH)])

    return deg_kernel


def _make_agg(N, E, D, feature_split):
    NR = E // KE
    if feature_split:
        WR = NR // NSUB
        P = 16
    else:
        WR = NR // (NCORES * NSUB)
        P = 8
    nph = WR // P
    CH = 40
    NCHUNK = N // CH
    nin = (P // 2) - 1
    ins = 2 if feature_split else 1

    @functools.partial(
        pl.kernel,
        out_type=jax.ShapeDtypeStruct((NCORES * N, D), jnp.float32),
        mesh=_mesh(),
        scratch_types=[
            pltpu.VMEM((P, KE), jnp.int32),
            pltpu.VMEM((P, KE), jnp.int32),
            pltpu.VMEM((KE, D), jnp.float32),
            pltpu.VMEM((KE, D), jnp.float32),
            pltpu.VMEM((CH, D), jnp.float32),
            pltpu.VMEM_SHARED((N, D), jnp.float32),
            pltpu.SemaphoreType.DMA,
            pltpu.SemaphoreType.DMA,
            pltpu.SemaphoreType.DMA,
            pltpu.SemaphoreType.DMA,
        ],
    )
    def agg_kernel(*args):
        u_refs = args[:ins]
        row_hbm, col_hbm, out_hbm = args[ins:ins + 3]
        (rA, cA, buf0, buf1, zbuf, acc,
         gsem0, gsem1, ssem0, ssem1) = args[ins + 3:]
        c = lax.axis_index("c")
        s = lax.axis_index("s")
        wbase = (s * WR) if feature_split else ((s * NCORES + c) * WR)

        def fill(r, _):
            for j in range(D // LANES):
                zbuf[r, pl.ds(j * LANES, LANES)] = jnp.zeros((LANES,),
                                                             jnp.float32)
            return 0

        lax.fori_loop(0, CH, fill, 0)
        for i in range((NCHUNK + NSUB - 1) // NSUB):
            k = s + i * NSUB

            @pl.when(k < NCHUNK)
            def _():
                pltpu.sync_copy(zbuf, acc.at[pl.ds(k * CH, CH)])
        plsc.subcore_barrier()

        def run(u_ref):
            def outer(ph, _):
                pltpu.sync_copy(row_hbm.at[pl.ds(wbase + ph * P, P)], rA)
                pltpu.sync_copy(col_hbm.at[pl.ds(wbase + ph * P, P)], cA)

                def pair(j, _):
                    l0 = 2 * j
                    g0 = pltpu.async_copy(u_ref.at[rA.at[l0]], buf0, gsem0)
                    g1 = pltpu.async_copy(u_ref.at[rA.at[l0 + 1]], buf1,
                                          gsem1)
                    g0.wait()
                    s0 = pltpu.async_copy(buf0, acc.at[cA.at[l0]], ssem0,
                                          add=True)
                    g1.wait()
                    s1 = pltpu.async_copy(buf1, acc.at[cA.at[l0 + 1]],
                                          ssem1, add=True)
                    s0.wait()
                    s1.wait()
                    return 0

                lax.fori_loop(0, P // 2, pair, 0)
                return 0

            lax.fori_loop(0, nph, outer, 0)

        if feature_split:
            @pl.when(c == 0)
            def _():
                run(u_refs[0])

            @pl.when(c == 1)
            def _():
                run(u_refs[1])
        else:
            run(u_refs[0])

        plsc.subcore_barrier()
        for i in range((NCHUNK + NSUB - 1) // NSUB):
            k = s + i * NSUB

            @pl.when(k < NCHUNK)
            def _():
                pltpu.sync_copy(acc.at[pl.ds(k * CH, CH)],
                                out_hbm.at[pl.ds(c * N + k * CH, CH)])

    return agg_kernel


def _make_edge(N, E, D):
    Dw = D // 2
    NR = E // KE
    WR = NR // (NCORES * NSUB)

    @functools.partial(
        pl.kernel,
        out_type=jax.ShapeDtypeStruct((NR, KE, Dw), jnp.int32),
        mesh=_mesh(),
        scratch_types=[
            pltpu.VMEM((WR, KE), jnp.int32),
            pltpu.VMEM((WR, KE), jnp.int32),
            pltpu.VMEM((KE, Dw), jnp.int32),
            pltpu.VMEM((KE, Dw), jnp.int32),
            pltpu.VMEM((KE, Dw), jnp.int32),
            pltpu.VMEM((KE, Dw), jnp.int32),
            pltpu.VMEM((KE, Dw), jnp.int32),
            pltpu.VMEM((KE, Dw), jnp.int32),
            pltpu.SemaphoreType.DMA,
            pltpu.SemaphoreType.DMA,
            pltpu.SemaphoreType.DMA,
            pltpu.SemaphoreType.DMA,
        ],
    )
    def edge_kernel(gs_hbm, gd_hbm, row_hbm, col_hbm, out_hbm,
                    rslab, cslab, bs0, bd0, bs1, bd1, ob, ob2,
                    gsem0, gsem1, stsem, stsem2):
        c = lax.axis_index("c")
        s = lax.axis_index("s")
        w = s * NCORES + c
        base = w * WR

        pltpu.sync_copy(row_hbm.at[pl.ds(base, WR)], rslab)
        pltpu.sync_copy(col_hbm.at[pl.ds(base, WR)], cslab)

        def issue(q, bs, bd, gsem):
            d0 = pltpu.async_copy(gs_hbm.at[rslab.at[q]], bs, gsem)
            d1 = pltpu.async_copy(gd_hbm.at[cslab.at[q]], bd, gsem)
            return d0, d1

        def compute(bs, bd, ob):
            M = jnp.int32(-65536)

            def _f(v):
                return lax.bitcast_convert_type(v, jnp.float32)

            def _i(v):
                return lax.bitcast_convert_type(v, jnp.int32)

            def body(i, _):
                for j in range(Dw // LANES):
                    sl = pl.ds(j * LANES, LANES)
                    wa = bs[i, sl]
                    wb = bd[i, sl]
                    rlo = jnp.maximum(_f(wa << 16) + _f(wb << 16), 0.0)
                    rhi = jnp.maximum(_f(wa & M) + _f(wb & M), 0.0)
                    ob[i, sl] = (lax.shift_right_logical(_i(rlo), 16)
                                 | (_i(rhi) & M))
                return 0

            lax.fori_loop(0, KE, body, 0)

        def body(ii, _):
            q = 2 * ii
            gs0, gd0 = issue(q, bs0, bd0, gsem0)
            gs1, gd1 = issue(q + 1, bs1, bd1, gsem1)
            gs0.wait()
            gd0.wait()
            compute(bs0, bd0, ob)
            st0 = pltpu.async_copy(ob, out_hbm.at[base + q], stsem)
            gs1.wait()
            gd1.wait()
            compute(bs1, bd1, ob2)
            st1 = pltpu.async_copy(ob2, out_hbm.at[base + q + 1], stsem2)
            st0.wait()
            st1.wait()
            return 0

        lax.fori_loop(0, WR // 2, body, 0)

    return edge_kernel


def _tc_h0(x, W1, blk=2000):
    N, D_IN = x.shape
    HID = W1.shape[1]

    def body(x_ref, w_ref, o_ref):
        o_ref[...] = jnp.dot(x_ref[...], w_ref[...],
                             preferred_element_type=jnp.float32)

    return pl.pallas_call(
        body,
        grid=(N // blk,),
        in_specs=[
            pl.BlockSpec((blk, D_IN), lambda i: (i, 0)),
            pl.BlockSpec((D_IN, HID), lambda i: (0, 0)),
        ],
        out_specs=pl.BlockSpec((blk, HID), lambda i: (i, 0)),
        out_shape=jax.ShapeDtypeStruct((N, HID), jnp.float32),
    )(x, W1)


def _tc_scale(h0, degp, blk=2000):
    N, HID = h0.shape
    Dh = HID // 2

    def body(h_ref, degp_ref, dinv_ref, ua_ref, ub_ref):
        deg = degp_ref[0, :, 0:1] + degp_ref[1, :, 0:1] + 1.0
        dinv = lax.rsqrt(deg)
        u = h_ref[...] * dinv
        dinv_ref[...] = dinv
        ua_ref[...] = u[:, :Dh]
        ub_ref[...] = u[:, Dh:]

    return pl.pallas_call(
        body,
        grid=(N // blk,),
        in_specs=[
            pl.BlockSpec((blk, HID), lambda i: (i, 0)),
            pl.BlockSpec((2, blk, 128), lambda i: (0, i, 0)),
        ],
        out_specs=[
            pl.BlockSpec((blk, 1), lambda i: (i, 0)),
            pl.BlockSpec((blk, Dh), lambda i: (i, 0)),
            pl.BlockSpec((blk, Dh), lambda i: (i, 0)),
        ],
        out_shape=[
            jax.ShapeDtypeStruct((N, 1), jnp.float32),
            jax.ShapeDtypeStruct((N, Dh), jnp.float32),
            jax.ShapeDtypeStruct((N, Dh), jnp.float32),
        ],
    )(h0, degp)


def _tc_mid(agg1, u1a, u1b, dinv, b1, W2, blk=2000):
    _, N, Dh = agg1.shape
    HID = 2 * Dh
    OUT = W2.shape[1]

    def body(a_ref, ua_ref, ub_ref, dinv_ref, b1_ref, w2_ref, u2_ref):
        t = jnp.concatenate([a_ref[0] + ua_ref[...], a_ref[1] + ub_ref[...]],
                            axis=1)
        dinv = dinv_ref[...]
        h = jnp.maximum(t * dinv + b1_ref[...], 0.0)
        u2_ref[...] = jnp.dot(h, w2_ref[...],
                              preferred_element_type=jnp.float32) * dinv

    return pl.pallas_call(
        body,
        grid=(N // blk,),
        in_specs=[
            pl.BlockSpec((2, blk, Dh), lambda i: (0, i, 0)),
            pl.BlockSpec((blk, Dh), lambda i: (i, 0)),
            pl.BlockSpec((blk, Dh), lambda i: (i, 0)),
            pl.BlockSpec((blk, 1), lambda i: (i, 0)),
            pl.BlockSpec((1, HID), lambda i: (0, 0)),
            pl.BlockSpec((HID, OUT), lambda i: (0, 0)),
        ],
        out_specs=pl.BlockSpec((blk, OUT), lambda i: (i, 0)),
        out_shape=jax.ShapeDtypeStruct((N, OUT), jnp.float32),
    )(agg1, u1a, u1b, dinv, b1, W2)


def _tc_node_out(agg2p, u2, dinv, b2, epW1, epb1, cW, cb, blk=2000):
    _, N, OUT = agg2p.shape
    EPH = epW1.shape[1]
    NC = cW.shape[1]

    def body(a_ref, u_ref, dinv_ref, b2_ref, w_ref, pb_ref, cw_ref, cb_ref,
             f_ref, gs_ref, gd_ref, lg_ref):
        t = a_ref[0] + a_ref[1] + u_ref[...]
        f = t * dinv_ref[...] + b2_ref[...]
        f_ref[...] = f
        w = w_ref[...]
        gs = jnp.dot(f, w[:OUT], preferred_element_type=jnp.float32) + pb_ref[...]
        gd = jnp.dot(f, w[OUT:], preferred_element_type=jnp.float32)
        gs_ref[...] = gs.astype(jnp.bfloat16)
        gd_ref[...] = gd.astype(jnp.bfloat16)
        lg_ref[...] = jnp.dot(f, cw_ref[...],
                              preferred_element_type=jnp.float32) + cb_ref[...]

    return pl.pallas_call(
        body,
        grid=(N // blk,),
        in_specs=[
            pl.BlockSpec((2, blk, OUT), lambda i: (0, i, 0)),
            pl.BlockSpec((blk, OUT), lambda i: (i, 0)),
            pl.BlockSpec((blk, 1), lambda i: (i, 0)),
            pl.BlockSpec((1, OUT), lambda i: (0, 0)),
            pl.BlockSpec((2 * OUT, EPH), lambda i: (0, 0)),
            pl.BlockSpec((1, EPH), lambda i: (0, 0)),
            pl.BlockSpec((OUT, NC), lambda i: (0, 0)),
            pl.BlockSpec((1, NC), lambda i: (0, 0)),
        ],
        out_specs=[
            pl.BlockSpec((blk, OUT), lambda i: (i, 0)),
            pl.BlockSpec((blk, EPH), lambda i: (i, 0)),
            pl.BlockSpec((blk, EPH), lambda i: (i, 0)),
            pl.BlockSpec((blk, NC), lambda i: (i, 0)),
        ],
        out_shape=[
            jax.ShapeDtypeStruct((N, OUT), jnp.float32),
            jax.ShapeDtypeStruct((N, EPH), jnp.bfloat16),
            jax.ShapeDtypeStruct((N, EPH), jnp.bfloat16),
            jax.ShapeDtypeStruct((N, NC), jnp.float32),
        ],
    )(agg2p, u2, dinv, b2, epW1, epb1, cW, cb)


def _tc_ef(a32, epW2, epb2, E_total, off_blk, ef_prev=None, blk=2000):
    Eslice, EPW = a32.shape
    EPH = 2 * EPW
    OUT = epW2.shape[1]

    def body(a_ref, we_ref, wo_ref, b_ref, o_ref):
        w = a_ref[...]
        fe = lax.bitcast_convert_type(w << 16, jnp.float32)
        fo = lax.bitcast_convert_type(w & jnp.int32(-65536), jnp.float32)
        o_ref[...] = (
            jnp.dot(fe.astype(jnp.bfloat16), we_ref[...].astype(jnp.bfloat16),
                    preferred_element_type=jnp.float32)
            + jnp.dot(fo.astype(jnp.bfloat16), wo_ref[...].astype(jnp.bfloat16),
                      preferred_element_type=jnp.float32)
            + b_ref[...])

    in_specs = [
        pl.BlockSpec((blk, EPW), lambda i: (i, 0)),
        pl.BlockSpec((EPW, OUT), lambda i: (0, 0)),
        pl.BlockSpec((EPW, OUT), lambda i: (0, 0)),
        pl.BlockSpec((1, OUT), lambda i: (0, 0)),
    ]
    args = [a32, epW2[0::2], epW2[1::2], epb2]
    nblk = Eslice // blk
    off = off_blk

    def out_map(i, _off=off):
        return (i + _off, 0)

    kwargs = {}
    if ef_prev is not None:
        in_specs.append(pl.BlockSpec((8, OUT), lambda i: (0, 0)))
        args.append(ef_prev)
        kwargs["input_output_aliases"] = {4: 0}

        def body_alias(a_ref, we_ref, wo_ref, b_ref, efin_ref, o_ref):
            body(a_ref, we_ref, wo_ref, b_ref, o_ref)

        run = body_alias
    else:
        run = body
    return pl.pallas_call(
        run,
        grid=(nblk,),
        in_specs=in_specs,
        out_specs=pl.BlockSpec((blk, OUT), out_map),
        out_shape=jax.ShapeDtypeStruct((E_total, OUT), jnp.float32),
        **kwargs,
    )(*args)


def kernel(x, edge_index, W1, b1, W2, b2, epW1, epb1, epW2, epb2, cW, cb):
    N, D_IN = x.shape
    HID = W1.shape[1]
    OUT = W2.shape[1]
    EPH = epW1.shape[1]
    E = edge_index.shape[1]
    row2d = edge_index[0].reshape(E // KE, KE)
    col2d = edge_index[1].reshape(E // KE, KE)

    degp = _make_degree(N, E)(col2d)
    h0 = _tc_h0(x, W1)
    dinv, u1a, u1b = _tc_scale(h0, degp.reshape(2, N, 128))
    agg1 = _make_agg(N, E, HID // 2, True)(u1a, u1b, row2d, col2d)
    u2 = _tc_mid(agg1.reshape(2, N, HID // 2), u1a, u1b, dinv,
                 b1.reshape(1, HID), W2)
    agg2p = _make_agg(N, E, OUT, False)(u2, row2d, col2d)
    f, gs, gd, logits = _tc_node_out(agg2p.reshape(2, N, OUT), u2, dinv,
                                     b2.reshape(1, OUT), epW1,
                                     epb1.reshape(1, EPH), cW,
                                     cb.reshape(1, cW.shape[1]))
    gs32 = lax.bitcast_convert_type(gs.reshape(N, EPH // 2, 2), jnp.int32)
    gd32 = lax.bitcast_convert_type(gd.reshape(N, EPH // 2, 2), jnp.int32)
    S = 10
    NRh = (E // KE) // S
    Eh = NRh * KE
    epb2r = epb2.reshape(1, OUT)
    edge_k = _make_edge(N, Eh, EPH)
    ef = None
    for si in range(S):
        a_si = edge_k(gs32, gd32, row2d[si * NRh:(si + 1) * NRh],
                      col2d[si * NRh:(si + 1) * NRh])
        ef = _tc_ef(a_si.reshape(Eh, EPH // 2), epW2, epb2r, E,
                    si * (Eh // 2000), ef)
    return (f, ef, logits, edge_index)


# speedup vs baseline: 1.0089x; 1.0089x over previous
import functools

import jax
import jax.numpy as jnp
from jax import lax
from jax.experimental import pallas as pl
from jax.experimental.pallas import tpu as pltpu
from jax.experimental.pallas import tpu_sc as plsc

NCORES = 2
NSUB = 16
LANES = 16
KE = 125


def _mesh():
    return plsc.VectorSubcoreMesh(core_axis_name="c", subcore_axis_name="s")


def _drain(src, dst, sem):
    pltpu.make_async_copy(src, dst, sem).wait()


def _make_degree(N, E):
    NR = E // KE
    WR = NR // (NCORES * NSUB)
    CH = 40
    NCHUNK = N // CH

    @functools.partial(
        pl.kernel,
        out_type=jax.ShapeDtypeStruct((NCORES * N, 128), jnp.float32),
        mesh=_mesh(),
        scratch_types=[
            pltpu.VMEM((WR, KE), jnp.int32),
            pltpu.VMEM((KE, 128), jnp.float32),
            pltpu.VMEM((CH, 128), jnp.float32),
            pltpu.VMEM_SHARED((N, 128), jnp.float32),
            pltpu.SemaphoreType.DMA,
            pltpu.SemaphoreType.DMA,
        ],
    )
    def deg_kernel(col_hbm, out_hbm, slab, ones, zbuf, acc, sem, sem2):
        c = lax.axis_index("c")
        s = lax.axis_index("s")
        w = s * NCORES + c

        def fill(r, _):
            for j in range(128 // LANES):
                sl = pl.ds(j * LANES, LANES)
                ones[r % KE, sl] = jnp.ones((LANES,), jnp.float32)
                zbuf[r % CH, sl] = jnp.zeros((LANES,), jnp.float32)
            return 0

        lax.fori_loop(0, max(KE, CH), fill, 0)
        pltpu.sync_copy(col_hbm.at[pl.ds(w * WR, WR)], slab)
        for i in range((NCHUNK + NSUB - 1) // NSUB):
            k = s + i * NSUB

            @pl.when(k < NCHUNK)
            def _():
                pltpu.sync_copy(zbuf, acc.at[pl.ds(k * CH, CH)])
        plsc.subcore_barrier()

        def batch(bi, _):
            d0 = pltpu.async_copy(ones, acc.at[slab.at[2 * bi]], sem,
                                  add=True)
            d1 = pltpu.async_copy(ones, acc.at[slab.at[2 * bi + 1]], sem2,
                                  add=True)
            d0.wait()
            d1.wait()
            return 0

        lax.fori_loop(0, WR // 2, batch, 0)
        plsc.subcore_barrier()
        for i in range((NCHUNK + NSUB - 1) // NSUB):
            k = s + i * NSUB

            @pl.when(k < NCHUNK)
            def _():
                pltpu.sync_copy(acc.at[pl.ds(k * CH, CH)],
                                out_hbm.at[pl.ds(c * N + k * CH, CH)])

    return deg_kernel


def _make_agg(N, E, D, feature_split):
    NR = E // KE
    if feature_split:
        WR = NR // NSUB
        P = 16
    else:
        WR = NR // (NCORES * NSUB)
        P = 8
    nph = WR // P
    CH = 40
    NCHUNK = N // CH
    nin = (P // 2) - 1
    ins = 2 if feature_split else 1

    @functools.partial(
        pl.kernel,
        out_type=jax.ShapeDtypeStruct((NCORES * N, D), jnp.float32),
        mesh=_mesh(),
        scratch_types=[
            pltpu.VMEM((P, KE), jnp.int32),
            pltpu.VMEM((P, KE), jnp.int32),
            pltpu.VMEM((KE, D), jnp.float32),
            pltpu.VMEM((KE, D), jnp.float32),
            pltpu.VMEM((CH, D), jnp.float32),
            pltpu.VMEM_SHARED((N, D), jnp.float32),
            pltpu.SemaphoreType.DMA,
            pltpu.SemaphoreType.DMA,
            pltpu.SemaphoreType.DMA,
            pltpu.SemaphoreType.DMA,
        ],
    )
    def agg_kernel(*args):
        u_refs = args[:ins]
        row_hbm, col_hbm, out_hbm = args[ins:ins + 3]
        (rA, cA, buf0, buf1, zbuf, acc,
         gsem0, gsem1, ssem0, ssem1) = args[ins + 3:]
        c = lax.axis_index("c")
        s = lax.axis_index("s")
        wbase = (s * WR) if feature_split else ((s * NCORES + c) * WR)

        def fill(r, _):
            for j in range(D // LANES):
                zbuf[r, pl.ds(j * LANES, LANES)] = jnp.zeros((LANES,),
                                                             jnp.float32)
            return 0

        lax.fori_loop(0, CH, fill, 0)
        for i in range((NCHUNK + NSUB - 1) // NSUB):
            k = s + i * NSUB

            @pl.when(k < NCHUNK)
            def _():
                pltpu.sync_copy(zbuf, acc.at[pl.ds(k * CH, CH)])
        plsc.subcore_barrier()

        def run(u_ref):
            def outer(ph, _):
                pltpu.sync_copy(row_hbm.at[pl.ds(wbase + ph * P, P)], rA)
                pltpu.sync_copy(col_hbm.at[pl.ds(wbase + ph * P, P)], cA)

                def pair(j, _):
                    l0 = 2 * j
                    g0 = pltpu.async_copy(u_ref.at[rA.at[l0]], buf0, gsem0)
                    g1 = pltpu.async_copy(u_ref.at[rA.at[l0 + 1]], buf1,
                                          gsem1)
                    g0.wait()
                    s0 = pltpu.async_copy(buf0, acc.at[cA.at[l0]], ssem0,
                                          add=True)
                    g1.wait()
                    s1 = pltpu.async_copy(buf1, acc.at[cA.at[l0 + 1]],
                                          ssem1, add=True)
                    s0.wait()
                    s1.wait()
                    return 0

                lax.fori_loop(0, P // 2, pair, 0)
                return 0

            lax.fori_loop(0, nph, outer, 0)

        if feature_split:
            @pl.when(c == 0)
            def _():
                run(u_refs[0])

            @pl.when(c == 1)
            def _():
                run(u_refs[1])
        else:
            run(u_refs[0])

        plsc.subcore_barrier()
        for i in range((NCHUNK + NSUB - 1) // NSUB):
            k = s + i * NSUB

            @pl.when(k < NCHUNK)
            def _():
                pltpu.sync_copy(acc.at[pl.ds(k * CH, CH)],
                                out_hbm.at[pl.ds(c * N + k * CH, CH)])

    return agg_kernel


def _make_edge(N, E, D):
    Dw = D // 2
    NR = E // KE
    WR = NR // (NCORES * NSUB)

    @functools.partial(
        pl.kernel,
        out_type=jax.ShapeDtypeStruct((NR, KE, Dw), jnp.int32),
        mesh=_mesh(),
        scratch_types=[
            pltpu.VMEM((WR, KE), jnp.int32),
            pltpu.VMEM((WR, KE), jnp.int32),
            pltpu.VMEM((KE, Dw), jnp.int32),
            pltpu.VMEM((KE, Dw), jnp.int32),
            pltpu.VMEM((KE, Dw), jnp.int32),
            pltpu.VMEM((KE, Dw), jnp.int32),
            pltpu.VMEM((KE, Dw), jnp.int32),
            pltpu.VMEM((KE, Dw), jnp.int32),
            pltpu.SemaphoreType.DMA,
            pltpu.SemaphoreType.DMA,
            pltpu.SemaphoreType.DMA,
            pltpu.SemaphoreType.DMA,
        ],
    )
    def edge_kernel(gs_hbm, gd_hbm, row_hbm, col_hbm, out_hbm,
                    rslab, cslab, bs0, bd0, bs1, bd1, ob, ob2,
                    gsem0, gsem1, stsem, stsem2):
        c = lax.axis_index("c")
        s = lax.axis_index("s")
        w = s * NCORES + c
        base = w * WR

        pltpu.sync_copy(row_hbm.at[pl.ds(base, WR)], rslab)
        pltpu.sync_copy(col_hbm.at[pl.ds(base, WR)], cslab)

        def issue(q, bs, bd, gsem):
            d0 = pltpu.async_copy(gs_hbm.at[rslab.at[q]], bs, gsem)
            d1 = pltpu.async_copy(gd_hbm.at[cslab.at[q]], bd, gsem)
            return d0, d1

        def compute(bs, bd, ob):
            M = jnp.int32(-65536)

            def _f(v):
                return lax.bitcast_convert_type(v, jnp.float32)

            def _i(v):
                return lax.bitcast_convert_type(v, jnp.int32)

            def body(i, _):
                for j in range(Dw // LANES):
                    sl = pl.ds(j * LANES, LANES)
                    wa = bs[i, sl]
                    wb = bd[i, sl]
                    rlo = jnp.maximum(_f(wa << 16) + _f(wb << 16), 0.0)
                    rhi = jnp.maximum(_f(wa & M) + _f(wb & M), 0.0)
                    ob[i, sl] = (lax.shift_right_logical(_i(rlo), 16)
                                 | (_i(rhi) & M))
                return 0

            lax.fori_loop(0, KE, body, 0)

        def body(ii, _):
            q = 2 * ii
            gs0, gd0 = issue(q, bs0, bd0, gsem0)
            gs1, gd1 = issue(q + 1, bs1, bd1, gsem1)
            gs0.wait()
            gd0.wait()
            compute(bs0, bd0, ob)
            st0 = pltpu.async_copy(ob, out_hbm.at[base + q], stsem)
            gs1.wait()
            gd1.wait()
            compute(bs1, bd1, ob2)
            st1 = pltpu.async_copy(ob2, out_hbm.at[base + q + 1], stsem2)
            st0.wait()
            st1.wait()
            return 0

        lax.fori_loop(0, WR // 2, body, 0)

    return edge_kernel


def _tc_h0(x, W1, blk=2000):
    N, D_IN = x.shape
    HID = W1.shape[1]

    def body(x_ref, w_ref, o_ref):
        o_ref[...] = jnp.dot(x_ref[...], w_ref[...],
                             preferred_element_type=jnp.float32)

    return pl.pallas_call(
        body,
        grid=(N // blk,),
        in_specs=[
            pl.BlockSpec((blk, D_IN), lambda i: (i, 0)),
            pl.BlockSpec((D_IN, HID), lambda i: (0, 0)),
        ],
        out_specs=pl.BlockSpec((blk, HID), lambda i: (i, 0)),
        out_shape=jax.ShapeDtypeStruct((N, HID), jnp.float32),
    )(x, W1)


def _tc_scale(x, W1, degp, blk=2000):
    N, D_IN = x.shape
    HID = W1.shape[1]
    Dh = HID // 2

    def body(x_ref, w_ref, degp_ref, dinv_ref, ua_ref, ub_ref):
        deg = degp_ref[0, :, 0:1] + degp_ref[1, :, 0:1] + 1.0
        dinv = lax.rsqrt(deg)
        h = jnp.dot(x_ref[...], w_ref[...], preferred_element_type=jnp.float32)
        u = h * dinv
        dinv_ref[...] = dinv
        ua_ref[...] = u[:, :Dh]
        ub_ref[...] = u[:, Dh:]

    return pl.pallas_call(
        body,
        grid=(N // blk,),
        in_specs=[
            pl.BlockSpec((blk, D_IN), lambda i: (i, 0)),
            pl.BlockSpec((D_IN, HID), lambda i: (0, 0)),
            pl.BlockSpec((2, blk, 128), lambda i: (0, i, 0)),
        ],
        out_specs=[
            pl.BlockSpec((blk, 1), lambda i: (i, 0)),
            pl.BlockSpec((blk, Dh), lambda i: (i, 0)),
            pl.BlockSpec((blk, Dh), lambda i: (i, 0)),
        ],
        out_shape=[
            jax.ShapeDtypeStruct((N, 1), jnp.float32),
            jax.ShapeDtypeStruct((N, Dh), jnp.float32),
            jax.ShapeDtypeStruct((N, Dh), jnp.float32),
        ],
    )(x, W1, degp)


def _tc_mid(agg1, u1a, u1b, dinv, b1, W2, blk=2000):
    _, N, Dh = agg1.shape
    HID = 2 * Dh
    OUT = W2.shape[1]

    def body(a_ref, ua_ref, ub_ref, dinv_ref, b1_ref, w2_ref, u2_ref):
        t = jnp.concatenate([a_ref[0] + ua_ref[...], a_ref[1] + ub_ref[...]],
                            axis=1)
        dinv = dinv_ref[...]
        h = jnp.maximum(t * dinv + b1_ref[...], 0.0)
        u2_ref[...] = jnp.dot(h, w2_ref[...],
                              preferred_element_type=jnp.float32) * dinv

    return pl.pallas_call(
        body,
        grid=(N // blk,),
        in_specs=[
            pl.BlockSpec((2, blk, Dh), lambda i: (0, i, 0)),
            pl.BlockSpec((blk, Dh), lambda i: (i, 0)),
            pl.BlockSpec((blk, Dh), lambda i: (i, 0)),
            pl.BlockSpec((blk, 1), lambda i: (i, 0)),
            pl.BlockSpec((1, HID), lambda i: (0, 0)),
            pl.BlockSpec((HID, OUT), lambda i: (0, 0)),
        ],
        out_specs=pl.BlockSpec((blk, OUT), lambda i: (i, 0)),
        out_shape=jax.ShapeDtypeStruct((N, OUT), jnp.float32),
    )(agg1, u1a, u1b, dinv, b1, W2)


def _tc_node_out(agg2p, u2, dinv, b2, epW1, epb1, cW, cb, blk=2000):
    _, N, OUT = agg2p.shape
    EPH = epW1.shape[1]
    NC = cW.shape[1]

    def body(a_ref, u_ref, dinv_ref, b2_ref, w_ref, pb_ref, cw_ref, cb_ref,
             f_ref, gs_ref, gd_ref, lg_ref):
        t = a_ref[0] + a_ref[1] + u_ref[...]
        f = t * dinv_ref[...] + b2_ref[...]
        f_ref[...] = f
        w = w_ref[...]
        gs = jnp.dot(f, w[:OUT], preferred_element_type=jnp.float32) + pb_ref[...]
        gd = jnp.dot(f, w[OUT:], preferred_element_type=jnp.float32)
        gs_ref[...] = gs.astype(jnp.bfloat16)
        gd_ref[...] = gd.astype(jnp.bfloat16)
        lg_ref[...] = jnp.dot(f, cw_ref[...],
                              preferred_element_type=jnp.float32) + cb_ref[...]

    return pl.pallas_call(
        body,
        grid=(N // blk,),
        in_specs=[
            pl.BlockSpec((2, blk, OUT), lambda i: (0, i, 0)),
            pl.BlockSpec((blk, OUT), lambda i: (i, 0)),
            pl.BlockSpec((blk, 1), lambda i: (i, 0)),
            pl.BlockSpec((1, OUT), lambda i: (0, 0)),
            pl.BlockSpec((2 * OUT, EPH), lambda i: (0, 0)),
            pl.BlockSpec((1, EPH), lambda i: (0, 0)),
            pl.BlockSpec((OUT, NC), lambda i: (0, 0)),
            pl.BlockSpec((1, NC), lambda i: (0, 0)),
        ],
        out_specs=[
            pl.BlockSpec((blk, OUT), lambda i: (i, 0)),
            pl.BlockSpec((blk, EPH), lambda i: (i, 0)),
            pl.BlockSpec((blk, EPH), lambda i: (i, 0)),
            pl.BlockSpec((blk, NC), lambda i: (i, 0)),
        ],
        out_shape=[
            jax.ShapeDtypeStruct((N, OUT), jnp.float32),
            jax.ShapeDtypeStruct((N, EPH), jnp.bfloat16),
            jax.ShapeDtypeStruct((N, EPH), jnp.bfloat16),
            jax.ShapeDtypeStruct((N, NC), jnp.float32),
        ],
    )(agg2p, u2, dinv, b2, epW1, epb1, cW, cb)


def _tc_ef(a32, epW2, epb2, E_total, off_blk, ef_prev=None, blk=2000):
    Eslice, EPW = a32.shape
    EPH = 2 * EPW
    OUT = epW2.shape[1]

    def body(a_ref, we_ref, wo_ref, b_ref, o_ref):
        w = a_ref[...]
        fe = lax.bitcast_convert_type(w << 16, jnp.float32)
        fo = lax.bitcast_convert_type(w & jnp.int32(-65536), jnp.float32)
        o_ref[...] = (
            jnp.dot(fe.astype(jnp.bfloat16), we_ref[...].astype(jnp.bfloat16),
                    preferred_element_type=jnp.float32)
            + jnp.dot(fo.astype(jnp.bfloat16), wo_ref[...].astype(jnp.bfloat16),
                      preferred_element_type=jnp.float32)
            + b_ref[...])

    in_specs = [
        pl.BlockSpec((blk, EPW), lambda i: (i, 0)),
        pl.BlockSpec((EPW, OUT), lambda i: (0, 0)),
        pl.BlockSpec((EPW, OUT), lambda i: (0, 0)),
        pl.BlockSpec((1, OUT), lambda i: (0, 0)),
    ]
    args = [a32, epW2[0::2], epW2[1::2], epb2]
    nblk = Eslice // blk
    off = off_blk

    def out_map(i, _off=off):
        return (i + _off, 0)

    kwargs = {}
    if ef_prev is not None:
        in_specs.append(pl.BlockSpec((8, OUT), lambda i: (0, 0)))
        args.append(ef_prev)
        kwargs["input_output_aliases"] = {4: 0}

        def body_alias(a_ref, we_ref, wo_ref, b_ref, efin_ref, o_ref):
            body(a_ref, we_ref, wo_ref, b_ref, o_ref)

        run = body_alias
    else:
        run = body
    return pl.pallas_call(
        run,
        grid=(nblk,),
        in_specs=in_specs,
        out_specs=pl.BlockSpec((blk, OUT), out_map),
        out_shape=jax.ShapeDtypeStruct((E_total, OUT), jnp.float32),
        **kwargs,
    )(*args)


def kernel(x, edge_index, W1, b1, W2, b2, epW1, epb1, epW2, epb2, cW, cb):
    N, D_IN = x.shape
    HID = W1.shape[1]
    OUT = W2.shape[1]
    EPH = epW1.shape[1]
    E = edge_index.shape[1]
    row2d = edge_index[0].reshape(E // KE, KE)
    col2d = edge_index[1].reshape(E // KE, KE)

    degp = _make_degree(N, E)(col2d)
    dinv, u1a, u1b = _tc_scale(x, W1, degp.reshape(2, N, 128))
    agg1 = _make_agg(N, E, HID // 2, True)(u1a, u1b, row2d, col2d)
    u2 = _tc_mid(agg1.reshape(2, N, HID // 2), u1a, u1b, dinv,
                 b1.reshape(1, HID), W2)
    agg2p = _make_agg(N, E, OUT, False)(u2, row2d, col2d)
    f, gs, gd, logits = _tc_node_out(agg2p.reshape(2, N, OUT), u2, dinv,
                                     b2.reshape(1, OUT), epW1,
                                     epb1.reshape(1, EPH), cW,
                                     cb.reshape(1, cW.shape[1]))
    gs32 = lax.bitcast_convert_type(gs.reshape(N, EPH // 2, 2), jnp.int32)
    gd32 = lax.bitcast_convert_type(gd.reshape(N, EPH // 2, 2), jnp.int32)
    S = 5
    NRh = (E // KE) // S
    Eh = NRh * KE
    epb2r = epb2.reshape(1, OUT)
    edge_k = _make_edge(N, Eh, EPH)
    ef = None
    for si in range(S):
        a_si = edge_k(gs32, gd32, row2d[si * NRh:(si + 1) * NRh],
                      col2d[si * NRh:(si + 1) * NRh])
        ef = _tc_ef(a_si.reshape(Eh, EPH // 2), epW2, epb2r, E,
                    si * (Eh // 2000), ef)
    return (f, ef, logits, edge_index)


# speedup vs baseline: 1.1248x; 1.1148x over previous
import functools

import jax
import jax.numpy as jnp
from jax import lax
from jax.experimental import pallas as pl
from jax.experimental.pallas import tpu as pltpu
from jax.experimental.pallas import tpu_sc as plsc

NCORES = 2
NSUB = 16
LANES = 16
KE = 125


def _mesh():
    return plsc.VectorSubcoreMesh(core_axis_name="c", subcore_axis_name="s")


def _drain(src, dst, sem):
    pltpu.make_async_copy(src, dst, sem).wait()


def _make_degree(N, E):
    NR = E // KE
    WR = NR // (NCORES * NSUB)
    CH = 40
    NCHUNK = N // CH

    @functools.partial(
        pl.kernel,
        out_type=jax.ShapeDtypeStruct((NCORES * N, 128), jnp.float32),
        mesh=_mesh(),
        scratch_types=[
            pltpu.VMEM((WR, KE), jnp.int32),
            pltpu.VMEM((KE, 128), jnp.float32),
            pltpu.VMEM((CH, 128), jnp.float32),
            pltpu.VMEM_SHARED((N, 128), jnp.float32),
            pltpu.SemaphoreType.DMA,
            pltpu.SemaphoreType.DMA,
        ],
    )
    def deg_kernel(col_hbm, out_hbm, slab, ones, zbuf, acc, sem, sem2):
        c = lax.axis_index("c")
        s = lax.axis_index("s")
        w = s * NCORES + c

        def fill(r, _):
            for j in range(128 // LANES):
                sl = pl.ds(j * LANES, LANES)
                ones[r % KE, sl] = jnp.ones((LANES,), jnp.float32)
                zbuf[r % CH, sl] = jnp.zeros((LANES,), jnp.float32)
            return 0

        lax.fori_loop(0, max(KE, CH), fill, 0)
        pltpu.sync_copy(col_hbm.at[pl.ds(w * WR, WR)], slab)
        for i in range((NCHUNK + NSUB - 1) // NSUB):
            k = s + i * NSUB

            @pl.when(k < NCHUNK)
            def _():
                pltpu.sync_copy(zbuf, acc.at[pl.ds(k * CH, CH)])
        plsc.subcore_barrier()

        def batch(bi, _):
            d0 = pltpu.async_copy(ones, acc.at[slab.at[2 * bi]], sem,
                                  add=True)
            d1 = pltpu.async_copy(ones, acc.at[slab.at[2 * bi + 1]], sem2,
                                  add=True)
            d0.wait()
            d1.wait()
            return 0

        lax.fori_loop(0, WR // 2, batch, 0)
        plsc.subcore_barrier()
        for i in range((NCHUNK + NSUB - 1) // NSUB):
            k = s + i * NSUB

            @pl.when(k < NCHUNK)
            def _():
                pltpu.sync_copy(acc.at[pl.ds(k * CH, CH)],
                                out_hbm.at[pl.ds(c * N + k * CH, CH)])

    return deg_kernel


def _make_agg(N, E, D, feature_split):
    NR = E // KE
    if feature_split:
        WR = NR // NSUB
        P = 16
    else:
        WR = NR // (NCORES * NSUB)
        P = 8
    nph = WR // P
    CH = 40
    NCHUNK = N // CH
    nin = (P // 2) - 1
    ins = 2 if feature_split else 1

    @functools.partial(
        pl.kernel,
        out_type=jax.ShapeDtypeStruct((NCORES * N, D), jnp.float32),
        mesh=_mesh(),
        scratch_types=[
            pltpu.VMEM((P, KE), jnp.int32),
            pltpu.VMEM((P, KE), jnp.int32),
            pltpu.VMEM((KE, D), jnp.float32),
            pltpu.VMEM((KE, D), jnp.float32),
            pltpu.VMEM((CH, D), jnp.float32),
            pltpu.VMEM_SHARED((N, D), jnp.float32),
            pltpu.SemaphoreType.DMA,
            pltpu.SemaphoreType.DMA,
            pltpu.SemaphoreType.DMA,
            pltpu.SemaphoreType.DMA,
        ],
    )
    def agg_kernel(*args):
        u_refs = args[:ins]
        row_hbm, col_hbm, out_hbm = args[ins:ins + 3]
        (rA, cA, buf0, buf1, zbuf, acc,
         gsem0, gsem1, ssem0, ssem1) = args[ins + 3:]
        c = lax.axis_index("c")
        s = lax.axis_index("s")
        wbase = (s * WR) if feature_split else ((s * NCORES + c) * WR)

        def fill(r, _):
            for j in range(D // LANES):
                zbuf[r, pl.ds(j * LANES, LANES)] = jnp.zeros((LANES,),
                                                             jnp.float32)
            return 0

        lax.fori_loop(0, CH, fill, 0)
        for i in range((NCHUNK + NSUB - 1) // NSUB):
            k = s + i * NSUB

            @pl.when(k < NCHUNK)
            def _():
                pltpu.sync_copy(zbuf, acc.at[pl.ds(k * CH, CH)])
        plsc.subcore_barrier()

        def run(u_ref):
            def outer(ph, _):
                pltpu.sync_copy(row_hbm.at[pl.ds(wbase + ph * P, P)], rA)
                pltpu.sync_copy(col_hbm.at[pl.ds(wbase + ph * P, P)], cA)

                def pair(j, _):
                    l0 = 2 * j
                    g0 = pltpu.async_copy(u_ref.at[rA.at[l0]], buf0, gsem0)
                    g1 = pltpu.async_copy(u_ref.at[rA.at[l0 + 1]], buf1,
                                          gsem1)
                    g0.wait()
                    s0 = pltpu.async_copy(buf0, acc.at[cA.at[l0]], ssem0,
                                          add=True)
                    g1.wait()
                    s1 = pltpu.async_copy(buf1, acc.at[cA.at[l0 + 1]],
                                          ssem1, add=True)
                    s0.wait()
                    s1.wait()
                    return 0

                lax.fori_loop(0, P // 2, pair, 0)
                return 0

            lax.fori_loop(0, nph, outer, 0)

        if feature_split:
            @pl.when(c == 0)
            def _():
                run(u_refs[0])

            @pl.when(c == 1)
            def _():
                run(u_refs[1])
        else:
            run(u_refs[0])

        plsc.subcore_barrier()
        for i in range((NCHUNK + NSUB - 1) // NSUB):
            k = s + i * NSUB

            @pl.when(k < NCHUNK)
            def _():
                pltpu.sync_copy(acc.at[pl.ds(k * CH, CH)],
                                out_hbm.at[pl.ds(c * N + k * CH, CH)])

    return agg_kernel


def _make_edge(N, E, D):
    Dw = D // 2
    NR = E // KE
    WR = NR // (NCORES * NSUB)

    @functools.partial(
        pl.kernel,
        out_type=jax.ShapeDtypeStruct((NR, KE, Dw), jnp.int32),
        mesh=_mesh(),
        scratch_types=[
            pltpu.VMEM((WR, KE), jnp.int32),
            pltpu.VMEM((WR, KE), jnp.int32),
            pltpu.VMEM((KE, Dw), jnp.int32),
            pltpu.VMEM((KE, Dw), jnp.int32),
            pltpu.VMEM((KE, Dw), jnp.int32),
            pltpu.VMEM((KE, Dw), jnp.int32),
            pltpu.VMEM((KE, Dw), jnp.int32),
            pltpu.VMEM((KE, Dw), jnp.int32),
            pltpu.SemaphoreType.DMA,
            pltpu.SemaphoreType.DMA,
            pltpu.SemaphoreType.DMA,
            pltpu.SemaphoreType.DMA,
        ],
    )
    def edge_kernel(gs_hbm, gd_hbm, row_hbm, col_hbm, out_hbm,
                    rslab, cslab, bs0, bd0, bs1, bd1, ob, ob2,
                    gsem0, gsem1, stsem, stsem2):
        c = lax.axis_index("c")
        s = lax.axis_index("s")
        w = s * NCORES + c
        base = w * WR

        pltpu.sync_copy(row_hbm.at[pl.ds(base, WR)], rslab)
        pltpu.sync_copy(col_hbm.at[pl.ds(base, WR)], cslab)

        def issue(q, bs, bd, gsem):
            d0 = pltpu.async_copy(gs_hbm.at[rslab.at[q]], bs, gsem)
            d1 = pltpu.async_copy(gd_hbm.at[cslab.at[q]], bd, gsem)
            return d0, d1

        def compute(bs, bd, ob):
            M = jnp.int32(-65536)

            def _f(v):
                return lax.bitcast_convert_type(v, jnp.float32)

            def _i(v):
                return lax.bitcast_convert_type(v, jnp.int32)

            def body(i, _):
                for j in range(Dw // LANES):
                    sl = pl.ds(j * LANES, LANES)
                    wa = bs[i, sl]
                    wb = bd[i, sl]
                    rlo = jnp.maximum(_f(wa << 16) + _f(wb << 16), 0.0)
                    rhi = jnp.maximum(_f(wa & M) + _f(wb & M), 0.0)
                    ob[i, sl] = (lax.shift_right_logical(_i(rlo), 16)
                                 | (_i(rhi) & M))
                return 0

            lax.fori_loop(0, KE, body, 0)

        def body(ii, _):
            q = 2 * ii
            gs0, gd0 = issue(q, bs0, bd0, gsem0)
            gs1, gd1 = issue(q + 1, bs1, bd1, gsem1)
            gs0.wait()
            gd0.wait()
            compute(bs0, bd0, ob)
            st0 = pltpu.async_copy(ob, out_hbm.at[base + q], stsem)
            gs1.wait()
            gd1.wait()
            compute(bs1, bd1, ob2)
            st1 = pltpu.async_copy(ob2, out_hbm.at[base + q + 1], stsem2)
            st0.wait()
            st1.wait()
            return 0

        lax.fori_loop(0, WR // 2, body, 0)

    return edge_kernel


def _tc_h0(x, W1, blk=2000):
    N, D_IN = x.shape
    HID = W1.shape[1]

    def body(x_ref, w_ref, o_ref):
        o_ref[...] = jnp.dot(x_ref[...], w_ref[...],
                             preferred_element_type=jnp.float32)

    return pl.pallas_call(
        body,
        grid=(N // blk,),
        in_specs=[
            pl.BlockSpec((blk, D_IN), lambda i: (i, 0)),
            pl.BlockSpec((D_IN, HID), lambda i: (0, 0)),
        ],
        out_specs=pl.BlockSpec((blk, HID), lambda i: (i, 0)),
        out_shape=jax.ShapeDtypeStruct((N, HID), jnp.float32),
    )(x, W1)


def _tc_scale(x, degp, blk=2000):
    N, D_IN = x.shape

    def body(x_ref, degp_ref, dinv_ref, v_ref):
        deg = degp_ref[0, :, 0:1] + degp_ref[1, :, 0:1] + 1.0
        dinv = lax.rsqrt(deg)
        dinv_ref[...] = dinv
        v_ref[...] = x_ref[...] * dinv

    return pl.pallas_call(
        body,
        grid=(N // blk,),
        in_specs=[
            pl.BlockSpec((blk, D_IN), lambda i: (i, 0)),
            pl.BlockSpec((2, blk, 128), lambda i: (0, i, 0)),
        ],
        out_specs=[
            pl.BlockSpec((blk, 1), lambda i: (i, 0)),
            pl.BlockSpec((blk, D_IN), lambda i: (i, 0)),
        ],
        out_shape=[
            jax.ShapeDtypeStruct((N, 1), jnp.float32),
            jax.ShapeDtypeStruct((N, D_IN), jnp.float32),
        ],
    )(x, degp)


def _tc_mid(agg1p, v, dinv, b1, W1, W2, blk=2000):
    _, N, D_IN = agg1p.shape
    HID = W1.shape[1]
    OUT = W2.shape[1]

    def body(a_ref, v_ref, dinv_ref, b1_ref, w1_ref, w2_ref, u2_ref):
        dinv = dinv_ref[...]
        t = (a_ref[0] + a_ref[1] + v_ref[...]) * dinv
        h = jnp.maximum(jnp.dot(t, w1_ref[...],
                                preferred_element_type=jnp.float32)
                        + b1_ref[...], 0.0)
        u2_ref[...] = jnp.dot(h, w2_ref[...],
                              preferred_element_type=jnp.float32) * dinv

    return pl.pallas_call(
        body,
        grid=(N // blk,),
        in_specs=[
            pl.BlockSpec((2, blk, D_IN), lambda i: (0, i, 0)),
            pl.BlockSpec((blk, D_IN), lambda i: (i, 0)),
            pl.BlockSpec((blk, 1), lambda i: (i, 0)),
            pl.BlockSpec((1, HID), lambda i: (0, 0)),
            pl.BlockSpec((D_IN, HID), lambda i: (0, 0)),
            pl.BlockSpec((HID, OUT), lambda i: (0, 0)),
        ],
        out_specs=pl.BlockSpec((blk, OUT), lambda i: (i, 0)),
        out_shape=jax.ShapeDtypeStruct((N, OUT), jnp.float32),
    )(agg1p, v, dinv, b1, W1, W2)


def _tc_node_out(agg2p, u2, dinv, b2, epW1, epb1, cW, cb, blk=2000):
    _, N, OUT = agg2p.shape
    EPH = epW1.shape[1]
    NC = cW.shape[1]

    def body(a_ref, u_ref, dinv_ref, b2_ref, w_ref, pb_ref, cw_ref, cb_ref,
             f_ref, gs_ref, gd_ref, lg_ref):
        t = a_ref[0] + a_ref[1] + u_ref[...]
        f = t * dinv_ref[...] + b2_ref[...]
        f_ref[...] = f
        w = w_ref[...]
        gs = jnp.dot(f, w[:OUT], preferred_element_type=jnp.float32) + pb_ref[...]
        gd = jnp.dot(f, w[OUT:], preferred_element_type=jnp.float32)
        gs_ref[...] = gs.astype(jnp.bfloat16)
        gd_ref[...] = gd.astype(jnp.bfloat16)
        lg_ref[...] = jnp.dot(f, cw_ref[...],
                              preferred_element_type=jnp.float32) + cb_ref[...]

    return pl.pallas_call(
        body,
        grid=(N // blk,),
        in_specs=[
            pl.BlockSpec((2, blk, OUT), lambda i: (0, i, 0)),
            pl.BlockSpec((blk, OUT), lambda i: (i, 0)),
            pl.BlockSpec((blk, 1), lambda i: (i, 0)),
            pl.BlockSpec((1, OUT), lambda i: (0, 0)),
            pl.BlockSpec((2 * OUT, EPH), lambda i: (0, 0)),
            pl.BlockSpec((1, EPH), lambda i: (0, 0)),
            pl.BlockSpec((OUT, NC), lambda i: (0, 0)),
            pl.BlockSpec((1, NC), lambda i: (0, 0)),
        ],
        out_specs=[
            pl.BlockSpec((blk, OUT), lambda i: (i, 0)),
            pl.BlockSpec((blk, EPH), lambda i: (i, 0)),
            pl.BlockSpec((blk, EPH), lambda i: (i, 0)),
            pl.BlockSpec((blk, NC), lambda i: (i, 0)),
        ],
        out_shape=[
            jax.ShapeDtypeStruct((N, OUT), jnp.float32),
            jax.ShapeDtypeStruct((N, EPH), jnp.bfloat16),
            jax.ShapeDtypeStruct((N, EPH), jnp.bfloat16),
            jax.ShapeDtypeStruct((N, NC), jnp.float32),
        ],
    )(agg2p, u2, dinv, b2, epW1, epb1, cW, cb)


def _tc_ef(a32, epW2, epb2, E_total, off_blk, ef_prev=None, blk=2000):
    Eslice, EPW = a32.shape
    EPH = 2 * EPW
    OUT = epW2.shape[1]

    def body(a_ref, we_ref, wo_ref, b_ref, o_ref):
        w = a_ref[...]
        fe = lax.bitcast_convert_type(w << 16, jnp.float32)
        fo = lax.bitcast_convert_type(w & jnp.int32(-65536), jnp.float32)
        o_ref[...] = (
            jnp.dot(fe.astype(jnp.bfloat16), we_ref[...].astype(jnp.bfloat16),
                    preferred_element_type=jnp.float32)
            + jnp.dot(fo.astype(jnp.bfloat16), wo_ref[...].astype(jnp.bfloat16),
                      preferred_element_type=jnp.float32)
            + b_ref[...])

    in_specs = [
        pl.BlockSpec((blk, EPW), lambda i: (i, 0)),
        pl.BlockSpec((EPW, OUT), lambda i: (0, 0)),
        pl.BlockSpec((EPW, OUT), lambda i: (0, 0)),
        pl.BlockSpec((1, OUT), lambda i: (0, 0)),
    ]
    args = [a32, epW2[0::2], epW2[1::2], epb2]
    nblk = Eslice // blk
    off = off_blk

    def out_map(i, _off=off):
        return (i + _off, 0)

    kwargs = {}
    if ef_prev is not None:
        in_specs.append(pl.BlockSpec((8, OUT), lambda i: (0, 0)))
        args.append(ef_prev)
        kwargs["input_output_aliases"] = {4: 0}

        def body_alias(a_ref, we_ref, wo_ref, b_ref, efin_ref, o_ref):
            body(a_ref, we_ref, wo_ref, b_ref, o_ref)

        run = body_alias
    else:
        run = body
    return pl.pallas_call(
        run,
        grid=(nblk,),
        in_specs=in_specs,
        out_specs=pl.BlockSpec((blk, OUT), out_map),
        out_shape=jax.ShapeDtypeStruct((E_total, OUT), jnp.float32),
        **kwargs,
    )(*args)


def kernel(x, edge_index, W1, b1, W2, b2, epW1, epb1, epW2, epb2, cW, cb):
    N, D_IN = x.shape
    HID = W1.shape[1]
    OUT = W2.shape[1]
    EPH = epW1.shape[1]
    E = edge_index.shape[1]
    row2d = edge_index[0].reshape(E // KE, KE)
    col2d = edge_index[1].reshape(E // KE, KE)

    degp = _make_degree(N, E)(col2d)
    dinv, v = _tc_scale(x, degp.reshape(2, N, 128))
    agg1p = _make_agg(N, E, D_IN, False)(v, row2d, col2d)
    u2 = _tc_mid(agg1p.reshape(2, N, D_IN), v, dinv,
                 b1.reshape(1, HID), W1, W2)
    agg2p = _make_agg(N, E, OUT, False)(u2, row2d, col2d)
    f, gs, gd, logits = _tc_node_out(agg2p.reshape(2, N, OUT), u2, dinv,
                                     b2.reshape(1, OUT), epW1,
                                     epb1.reshape(1, EPH), cW,
                                     cb.reshape(1, cW.shape[1]))
    gs32 = lax.bitcast_convert_type(gs.reshape(N, EPH // 2, 2), jnp.int32)
    gd32 = lax.bitcast_convert_type(gd.reshape(N, EPH // 2, 2), jnp.int32)
    S = 5
    NRh = (E // KE) // S
    Eh = NRh * KE
    epb2r = epb2.reshape(1, OUT)
    edge_k = _make_edge(N, Eh, EPH)
    ef = None
    for si in range(S):
        a_si = edge_k(gs32, gd32, row2d[si * NRh:(si + 1) * NRh],
                      col2d[si * NRh:(si + 1) * NRh])
        ef = _tc_ef(a_si.reshape(Eh, EPH // 2), epW2, epb2r, E,
                    si * (Eh // 2000), ef)
    return (f, ef, logits, edge_index)
